# Initial kernel scaffold; baseline (speedup 1.0000x reference)
import functools

import jax
import jax.numpy as jnp
from jax import lax
from jax.experimental import pallas as pl
from jax.experimental.pallas import tpu as pltpu
from jax.experimental.pallas import tpu_sc as plsc

NUM_ROWS = 400
EDIM = 32
XW = 16
OW = 78
CHUNK = 512


def _body(x_hbm, w_hbm, out_hbm, w_v, x_v, out_v, *, tokens_per_worker, num_cores):
    wid = lax.axis_index("s") * num_cores + lax.axis_index("c")
    pltpu.sync_copy(w_hbm, w_v)
    base0 = wid * tokens_per_worker
    n_chunks = tokens_per_worker // CHUNK

    def do_chunk(ci, _):
        base = base0 + ci * CHUNK
        pltpu.sync_copy(x_hbm.at[pl.ds(base, CHUNK)], x_v)

        def do_group(g, _):
            tok = g * 16 + lax.iota(jnp.int32, 16)

            def col(c):
                return jnp.full((16,), c, jnp.int32)

            va = plsc.load_gather(x_v, [tok, col(0)])
            ia = jnp.clip(va.astype(jnp.int32), 0, NUM_ROWS - 1)
            vo = plsc.load_gather(x_v, [tok, col(8)])
            io = jnp.clip(vo.astype(jnp.int32), 0, NUM_ROWS - 1)

            for c in range(EDIM):
                ea = plsc.load_gather(w_v, [ia, col(c)])
                plsc.store_scatter(out_v, [tok, col(c)], ea)
                eo = plsc.load_gather(w_v, [io, col(c)])
                plsc.store_scatter(out_v, [tok, col(39 + c)], eo)
            for c in range(7):
                sa = plsc.load_gather(x_v, [tok, col(1 + c)])
                plsc.store_scatter(out_v, [tok, col(32 + c)], sa)
                so = plsc.load_gather(x_v, [tok, col(9 + c)])
                plsc.store_scatter(out_v, [tok, col(71 + c)], so)
            return 0

        lax.fori_loop(0, CHUNK // 16, do_group, 0)
        pltpu.sync_copy(out_v, out_hbm.at[pl.ds(base, CHUNK)])
        return 0

    lax.fori_loop(0, n_chunks, do_chunk, 0)


def kernel(x, W):
    B, H, _ = x.shape
    n = B * H
    xf = x.reshape(n, XW)
    mesh = plsc.VectorSubcoreMesh(core_axis_name="c", subcore_axis_name="s")
    n_workers = mesh.num_cores * mesh.num_subcores
    tokens_per_worker = n // n_workers
    assert tokens_per_worker * n_workers == n
    assert tokens_per_worker % CHUNK == 0

    body = functools.partial(
        _body,
        tokens_per_worker=tokens_per_worker,
        num_cores=mesh.num_cores,
    )
    out = pl.kernel(
        body,
        out_type=jax.ShapeDtypeStruct((n, OW), jnp.float32),
        mesh=mesh,
        scratch_types=[
            pltpu.VMEM((NUM_ROWS, EDIM), jnp.float32),
            pltpu.VMEM((CHUNK, XW), jnp.float32),
            pltpu.VMEM((CHUNK, OW), jnp.float32),
        ],
    )(xf, W)
    return out.reshape(B, H, OW)


# speedup vs baseline: 2.6996x; 2.6996x over previous
import functools

import jax
import jax.numpy as jnp
from jax import lax
from jax.experimental import pallas as pl
from jax.experimental.pallas import tpu as pltpu
from jax.experimental.pallas import tpu_sc as plsc

NUM_ROWS = 400
EDIM = 32
XW = 16
OW = 78
CHUNK = 512


def _body(x_hbm, w_hbm, out_hbm, w_v, x_v, out_v, *, tokens_per_worker, num_cores):
    wid = lax.axis_index("s") * num_cores + lax.axis_index("c")
    pltpu.sync_copy(w_hbm, w_v)
    base0 = wid * tokens_per_worker
    n_chunks = tokens_per_worker // CHUNK

    def do_chunk(ci, _):
        base = base0 + ci * CHUNK
        pltpu.sync_copy(x_hbm.at[pl.ds(base, CHUNK)], x_v)

        def do_group(g, _):
            tok = g * 16 + lax.iota(jnp.int32, 16)

            def col(c):
                return jnp.full((16,), c, jnp.int32)

            va = plsc.load_gather(x_v, [tok, col(0)])
            ia = jnp.clip(va.astype(jnp.int32), 0, NUM_ROWS - 1)
            vo = plsc.load_gather(x_v, [tok, col(8)])
            io = jnp.clip(vo.astype(jnp.int32), 0, NUM_ROWS - 1)

            for c in range(EDIM):
                ea = plsc.load_gather(w_v, [ia, col(c)])
                plsc.store_scatter(out_v, [tok, col(c)], ea)
                eo = plsc.load_gather(w_v, [io, col(c)])
                plsc.store_scatter(out_v, [tok, col(39 + c)], eo)
            for c in range(7):
                sa = plsc.load_gather(x_v, [tok, col(1 + c)])
                plsc.store_scatter(out_v, [tok, col(32 + c)], sa)
                so = plsc.load_gather(x_v, [tok, col(9 + c)])
                plsc.store_scatter(out_v, [tok, col(71 + c)], so)
            return 0

        lax.fori_loop(0, CHUNK // 16, do_group, 0)
        pltpu.sync_copy(out_v, out_hbm.at[pl.ds(base, CHUNK)])
        return 0

    lax.fori_loop(0, n_chunks, do_chunk, 0)


def kernel(x, W):
    B, H, _ = x.shape
    n = B * H
    xf = x.reshape(n, XW)
    mesh = plsc.VectorSubcoreMesh(core_axis_name="c", subcore_axis_name="s")
    n_workers = mesh.num_cores * mesh.num_subcores
    tokens_per_worker = n // n_workers
    assert tokens_per_worker * n_workers == n
    assert tokens_per_worker % CHUNK == 0

    body = functools.partial(
        _body,
        tokens_per_worker=tokens_per_worker,
        num_cores=mesh.num_cores,
    )
    out = pl.kernel(
        body,
        out_type=jax.ShapeDtypeStruct((n, OW), jnp.float32),
        mesh=mesh,
        compiler_params=pltpu.CompilerParams(
            needs_layout_passes=False, use_tc_tiling_on_sc=False
        ),
        scratch_types=[
            pltpu.VMEM((NUM_ROWS, EDIM), jnp.float32),
            pltpu.VMEM((CHUNK, XW), jnp.float32),
            pltpu.VMEM((CHUNK, OW), jnp.float32),
        ],
    )(xf, W)
    return out.reshape(B, H, OW)


# speedup vs baseline: 2.8486x; 1.0552x over previous
import functools

import jax
import jax.numpy as jnp
from jax import lax
from jax.experimental import pallas as pl
from jax.experimental.pallas import tpu as pltpu
from jax.experimental.pallas import tpu_sc as plsc

NUM_ROWS = 400
EDIM = 32
XW = 16
OW = 78
CHUNK = 512


def _body(x_hbm, w_hbm, out_hbm, w_v, x_v, out_v, *, tokens_per_worker, num_cores):
    wid = lax.axis_index("s") * num_cores + lax.axis_index("c")
    pltpu.sync_copy(w_hbm, w_v)
    base0 = wid * tokens_per_worker
    n_chunks = tokens_per_worker // CHUNK

    def do_chunk(ci, _):
        base = base0 + ci * CHUNK
        pltpu.sync_copy(x_hbm.at[pl.ds(base, CHUNK)], x_v)

        @plsc.parallel_loop(0, CHUNK // 16, unroll=2)
        def do_group(g):
            tok = g * 16 + lax.iota(jnp.int32, 16)

            def col(c):
                return jnp.full((16,), c, jnp.int32)

            va = plsc.load_gather(x_v, [tok, col(0)])
            ia = jnp.clip(va.astype(jnp.int32), 0, NUM_ROWS - 1)
            vo = plsc.load_gather(x_v, [tok, col(8)])
            io = jnp.clip(vo.astype(jnp.int32), 0, NUM_ROWS - 1)

            for c in range(EDIM):
                ea = plsc.load_gather(w_v, [ia, col(c)])
                plsc.store_scatter(out_v, [tok, col(c)], ea)
                eo = plsc.load_gather(w_v, [io, col(c)])
                plsc.store_scatter(out_v, [tok, col(39 + c)], eo)
            for c in range(7):
                sa = plsc.load_gather(x_v, [tok, col(1 + c)])
                plsc.store_scatter(out_v, [tok, col(32 + c)], sa)
                so = plsc.load_gather(x_v, [tok, col(9 + c)])
                plsc.store_scatter(out_v, [tok, col(71 + c)], so)

        pltpu.sync_copy(out_v, out_hbm.at[pl.ds(base, CHUNK)])
        return 0

    lax.fori_loop(0, n_chunks, do_chunk, 0)


def kernel(x, W):
    B, H, _ = x.shape
    n = B * H
    xf = x.reshape(n, XW)
    mesh = plsc.VectorSubcoreMesh(core_axis_name="c", subcore_axis_name="s")
    n_workers = mesh.num_cores * mesh.num_subcores
    tokens_per_worker = n // n_workers
    assert tokens_per_worker * n_workers == n
    assert tokens_per_worker % CHUNK == 0

    body = functools.partial(
        _body,
        tokens_per_worker=tokens_per_worker,
        num_cores=mesh.num_cores,
    )
    out = pl.kernel(
        body,
        out_type=jax.ShapeDtypeStruct((n, OW), jnp.float32),
        mesh=mesh,
        compiler_params=pltpu.CompilerParams(
            needs_layout_passes=False,
            use_tc_tiling_on_sc=False,
            disable_bounds_checks=True,
        ),
        scratch_types=[
            pltpu.VMEM((NUM_ROWS, EDIM), jnp.float32),
            pltpu.VMEM((CHUNK, XW), jnp.float32),
            pltpu.VMEM((CHUNK, OW), jnp.float32),
        ],
    )(xf, W)
    return out.reshape(B, H, OW)
